# Initial kernel scaffold; baseline (speedup 1.0000x reference)
import jax
import jax.numpy as jnp
from jax.experimental import pallas as pl


def kernel(feat, segment_ids, W, b):
    raise NotImplementedError("write your pallas kernel here")


# speedup vs baseline: 17.7248x; 17.7248x over previous
import jax
import jax.numpy as jnp
from jax import lax
from jax.experimental import pallas as pl
from jax.experimental.pallas import tpu as pltpu
from jax.experimental.pallas import tpu_sc as plsc

_B = 128
_K = 30
_D = 256
_ST = 144
_BN = 2000

_NC, _NS, _L = 2, 16, 16
_NW = _NC * _NS
_GPW = _B // _NW
_BIGI = jnp.int32(2 ** 30)


def _tc_rowmax_starts(feat_ref, seg_ref, keys_ref, starts_ref):
    i = pl.program_id(0)
    f = feat_ref[...]
    keys_ref[...] = jnp.max(f, axis=1, keepdims=True)
    seg = seg_ref[0]
    t = lax.broadcasted_iota(jnp.int32, (_ST, 1), 0)
    cnt = jnp.sum((seg < t).astype(jnp.int32), axis=1, keepdims=True)

    @pl.when(i == 0)
    def _():
        starts_ref[...] = cnt

    @pl.when(i != 0)
    def _():
        starts_ref[...] = starts_ref[...] + cnt


def _sc_topk_gather(n_nodes, keys_hbm, starts_hbm, feat_hbm,
                    rows_hbm, valid_hbm,
                    keys_v, starts_v, idx_v, rows_v, valid_v, sem):
    c = lax.axis_index("c")
    s_ = lax.axis_index("s")
    wid = s_ * _NC + c
    pltpu.sync_copy(starts_hbm, starts_v)
    pltpu.sync_copy(keys_hbm, keys_v.at[pl.ds(0, n_nodes)])
    lanes = lax.iota(jnp.int32, _L)
    lane0 = lanes == 0
    neg = jnp.float32(-3.0e38)

    for gg in range(_GPW):
        g = wid * _GPW + gg
        sv = starts_v[pl.ds(g, _L)]
        s0 = sv[0]
        e0 = sv[1]
        span = e0 - s0
        nch = (span + (_L - 1)) // _L

        idx_v[pl.ds(0, _L)] = jnp.zeros((_L,), jnp.int32)
        idx_v[pl.ds(_L, _L)] = jnp.zeros((_L,), jnp.int32)

        def k_body(k, carry):
            pv, pi = carry

            def c_body(ci, bc):
                bv, bi = bc
                off = s0 + ci * _L
                v = keys_v[pl.ds(off, _L)]
                gi = off + lanes
                elig = (gi < e0) & ((v < pv) | ((v == pv) & (gi > pi)))
                cand = jnp.where(elig, v, neg)
                take = (cand > bv) | ((cand == bv) & (gi < bi))
                return (jnp.where(take, cand, bv), jnp.where(take, gi, bi))

            bv0 = jnp.full((_L,), neg, jnp.float32)
            bi0 = jnp.full((_L,), _BIGI, jnp.int32)
            bv, bi = lax.fori_loop(0, nch, c_body, (bv0, bi0))
            sv, si = plsc.sort_key_val(bv, bi, descending=True)
            m = sv[0]
            ii = jnp.where(sv == m, si, _BIGI)
            mi_vec, _ = plsc.sort_key_val(ii, ii)
            mi = mi_vec[0]
            row = jnp.where(k < span, mi, 0)
            plsc.store_scatter(idx_v, [jnp.full((_L,), k, jnp.int32)],
                               jnp.full((_L,), row, jnp.int32), mask=lane0)
            return (m, mi)

        lax.fori_loop(0, _K, k_body, (jnp.float32(3.0e38), jnp.int32(-1)))

        pltpu.async_copy(feat_hbm.at[idx_v], rows_v, sem).wait()
        pltpu.sync_copy(rows_v, rows_hbm.at[g])

        valid_v[pl.ds(0, _L)] = (lanes < span).astype(jnp.float32)
        valid_v[pl.ds(_L, _L)] = ((lanes + _L) < span).astype(jnp.float32)
        pltpu.sync_copy(valid_v, valid_hbm.at[g])


def _sort_cols(x):
    S, R = x.shape
    s = 2
    while s <= S:
        d = s // 2
        while d >= 1:
            G = S // (2 * d)
            xr = x.reshape(G, 2, d, R)
            a = xr[:, 0]
            b2 = xr[:, 1]
            lo = jnp.minimum(a, b2)
            hi = jnp.maximum(a, b2)
            gi = lax.broadcasted_iota(jnp.int32, (G, 1, 1), 0)
            asc = ((gi * (2 * d)) & s) == 0
            na = jnp.where(asc, lo, hi)
            nb = jnp.where(asc, hi, lo)
            x = jnp.stack([na, nb], axis=1).reshape(S, R)
            d //= 2
        s *= 2
    return x


def _tc_sort_matmul(rows_ref, w_ref, valid_ref, out_ref):
    k = pl.program_id(0)
    g = rows_ref[0]
    srt = _sort_cols(g.T) * valid_ref[0]
    prod = lax.dot_general(w_ref[...], srt, (((1,), (0,)), ((), ())),
                           preferred_element_type=jnp.float32)

    @pl.when(k == 0)
    def _():
        out_ref[...] = prod

    @pl.when(k != 0)
    def _():
        out_ref[...] = out_ref[...] + prod


def kernel(feat, segment_ids, W, b):
    n, d = feat.shape
    assert d == _D and n % _BN == 0 and n % _L == 0
    seg3 = segment_ids.astype(jnp.int32).reshape(n // _BN, 1, _BN)

    keys2, starts2 = pl.pallas_call(
        _tc_rowmax_starts,
        grid=(n // _BN,),
        in_specs=[pl.BlockSpec((_BN, _D), lambda i: (i, 0)),
                  pl.BlockSpec((1, 1, _BN), lambda i: (i, 0, 0))],
        out_specs=[pl.BlockSpec((_BN, 1), lambda i: (i, 0)),
                   pl.BlockSpec((_ST, 1), lambda i: (0, 0))],
        out_shape=[jax.ShapeDtypeStruct((n, 1), jnp.float32),
                   jax.ShapeDtypeStruct((_ST, 1), jnp.int32)],
    )(feat, seg3)

    keys1d = keys2.reshape(n)
    starts1d = starts2.reshape(_ST)

    mesh = plsc.VectorSubcoreMesh(core_axis_name="c", subcore_axis_name="s")
    rows, valid = pl.kernel(
        lambda *a: _sc_topk_gather(n, *a),
        mesh=mesh,
        compiler_params=pltpu.CompilerParams(needs_layout_passes=False),
        out_type=[jax.ShapeDtypeStruct((_B, 2 * _L, _D), jnp.float32),
                  jax.ShapeDtypeStruct((_B, 2 * _L), jnp.float32)],
        scratch_types=[
            pltpu.VMEM((n + _L,), jnp.float32),
            pltpu.VMEM((_ST,), jnp.int32),
            pltpu.VMEM((2 * _L,), jnp.int32),
            pltpu.VMEM((2 * _L, _D), jnp.float32),
            pltpu.VMEM((2 * _L,), jnp.float32),
            pltpu.SemaphoreType.DMA,
        ],
    )(keys1d, starts1d, feat)

    rows_kbd = rows.transpose(1, 0, 2)
    valid3 = valid.T.reshape(2 * _L, 1, _B)
    outT = pl.pallas_call(
        _tc_sort_matmul,
        grid=(_K,),
        in_specs=[pl.BlockSpec((1, _B, _D), lambda k: (k, 0, 0)),
                  pl.BlockSpec((_D, _D), lambda k: (0, k)),
                  pl.BlockSpec((1, 1, _B), lambda k: (k, 0, 0))],
        out_specs=pl.BlockSpec((_D, _B), lambda k: (0, 0)),
        out_shape=jax.ShapeDtypeStruct((_D, _B), jnp.float32),
    )(rows_kbd, W, valid3)

    return outT.T + b[None, :]
